# Initial kernel scaffold; baseline (speedup 1.0000x reference)
import jax
import jax.numpy as jnp
from jax.experimental import pallas as pl


def kernel(latent_embedding, codebook):
    raise NotImplementedError("write your pallas kernel here")


# speedup vs baseline: 8.9824x; 8.9824x over previous
import functools

import jax
import jax.numpy as jnp
from jax import lax
from jax.experimental import pallas as pl
from jax.experimental.pallas import tpu as pltpu
from jax.experimental.pallas import tpu_sc as plsc

_NUM_LATENT = 8192
_LATENT_DIM = 256
_BETA = 0.25
_BM = 256


def _tc_body(n_rows, z_ref, cb_ref, idx_ref, probs_ref, loss_ref, bsq_ref):
    i = pl.program_id(0)
    nb = pl.num_programs(0)
    cb = cb_ref[...]

    @pl.when(i == 0)
    def _init():
        bsq_ref[...] = jnp.sum(cb * cb, axis=1).reshape(1, _NUM_LATENT)

    z = z_ref[...]
    a = jnp.sum(z * z, axis=1, keepdims=True)
    m = lax.dot_general(
        z, cb, dimension_numbers=(((1,), (1,)), ((), ())),
        preferred_element_type=jnp.float32)
    d = (a + bsq_ref[...]) - 2.0 * m
    dmin = jnp.min(d, axis=1, keepdims=True)
    jf = lax.broadcasted_iota(jnp.int32, d.shape, 1).astype(jnp.float32)
    idxf = jnp.min(jnp.where(d == dmin, jf, jnp.float32(_NUM_LATENT)), axis=1)
    idx_ref[...] = idxf.astype(jnp.int32).reshape(1, 1, _BM)

    p = jnp.exp(dmin - d)
    s = jnp.sum(p, axis=1, keepdims=True)
    sm = p * (1.0 / s)
    colsum = jnp.sum(sm, axis=0, keepdims=True)
    tot = jnp.where(i == 0, colsum, probs_ref[...] + colsum)
    probs_ref[...] = jnp.where(
        i == nb - 1, jnp.clip(tot * (1.0 / n_rows), 0.001, 0.999), tot)

    ds = jnp.sum(dmin)
    lt = jnp.where(i == 0, ds, loss_ref[0, 0] + ds)
    scale = (1.0 + _BETA) * _NUM_LATENT / (n_rows * _LATENT_DIM)
    loss_ref[0, 0] = jnp.where(i == nb - 1, lt * scale, lt)


def _tc_call(latent_embedding, codebook):
    n = latent_embedding.shape[0]
    nb = n // _BM
    return pl.pallas_call(
        functools.partial(_tc_body, float(n)),
        grid=(nb,),
        in_specs=[
            pl.BlockSpec((_BM, _LATENT_DIM), lambda i: (i, 0)),
            pl.BlockSpec((_NUM_LATENT, _LATENT_DIM), lambda i: (0, 0)),
        ],
        out_specs=[
            pl.BlockSpec((1, 1, _BM), lambda i: (i, 0, 0)),
            pl.BlockSpec((1, _NUM_LATENT), lambda i: (0, 0)),
            pl.BlockSpec((1, 1), lambda i: (0, 0), memory_space=pltpu.SMEM),
        ],
        out_shape=[
            jax.ShapeDtypeStruct((nb, 1, _BM), jnp.int32),
            jax.ShapeDtypeStruct((1, _NUM_LATENT), jnp.float32),
            jax.ShapeDtypeStruct((1, 1), jnp.float32),
        ],
        scratch_shapes=[pltpu.VMEM((1, _NUM_LATENT), jnp.float32)],
    )(latent_embedding, codebook)


def _sc_gather(codebook, idx):
    b = idx.shape[0]
    d = codebook.shape[1]
    nc, ns = 2, 16
    nw = nc * ns
    bw = b // nw
    ch = 128
    nch = bw // ch
    mesh = plsc.VectorSubcoreMesh(
        core_axis_name="c", subcore_axis_name="s",
        num_cores=nc, num_subcores=ns)

    @functools.partial(
        pl.kernel, mesh=mesh,
        out_type=jax.ShapeDtypeStruct((b, d), jnp.float32),
        scratch_types=[
            pltpu.VMEM((ch,), jnp.int32),
            pltpu.VMEM((ch, d), jnp.float32),
            pltpu.SemaphoreType.DMA,
        ])
    def k(cb_hbm, idx_hbm, out_hbm, idx_v, rows_v, sem):
        wid = lax.axis_index("s") * nc + lax.axis_index("c")
        base = wid * bw
        for c in range(nch):
            o = base + c * ch
            pltpu.sync_copy(idx_hbm.at[pl.ds(o, ch)], idx_v)
            pltpu.async_copy(cb_hbm.at[idx_v], rows_v, sem).wait()
            pltpu.sync_copy(rows_v, out_hbm.at[pl.ds(o, ch)])

    return k(codebook, idx)


def kernel(latent_embedding, codebook):
    n = latent_embedding.shape[0]
    idx3, probs, loss = _tc_call(latent_embedding, codebook)
    idx = idx3.reshape(n)
    quantized = _sc_gather(codebook, idx)
    return (quantized, codebook, probs, loss.reshape(()))
